# baseline (speedup 1.0000x reference)
import jax
import jax.numpy as jnp
from jax import lax
from jax.experimental import pallas as pl
from jax.experimental.pallas import tpu as pltpu
from jax.experimental.pallas import tpu_sc as plsc

N_NODES = 10000
D_FEAT = 256
N_EDGES = 160000
NC, NS = 2, 16
NW = NC * NS
CPW = D_FEAT // NW
ECH = 640
NCH = N_EDGES // ECH
SUB = 128
NSUB = ECH // SUB
ACC = N_NODES * CPW

NEG_INF = float("-inf")

_GATHER_DNUMS = lax.GatherDimensionNumbers(
    offset_dims=(), collapsed_slice_dims=(0,), start_index_map=(0,))


def _swap_halves(v, perm):
    return lax.gather(v, perm, _GATHER_DNUMS, (1,),
                      mode=lax.GatherScatterMode.PROMISE_IN_BOUNDS)


def _body(xt, srcs, dsts, out, src_b, dst_b, vals, acc, scr,
          sl0, sl1, sg0, sg1):
    wid = lax.axis_index("c") * NS + lax.axis_index("s")
    lane = lax.iota(jnp.int32, 16)
    col8 = lane & 7
    half = lane >> 3
    perm = jnp.reshape(lane ^ 8, (16, 1))

    def init(i, carry):
        acc[pl.ds(i * 16, 16)] = jnp.full((16,), NEG_INF, jnp.float32)
        return carry
    lax.fori_loop(0, ACC // 16, init, 0, unroll=4)

    my_x = xt.at[wid]
    sem_l = (sl0, sl1)
    sem_g = (sg0, sg1)

    def issue_linear(c, s):
        pltpu.async_copy(srcs.at[c], src_b.at[pl.ds(s * NSUB, NSUB)], sem_l[s])
        pltpu.async_copy(dsts.at[c], dst_b.at[pl.ds(s * ECH, ECH)], sem_l[s])

    def wait_linear(s):
        pltpu.make_async_copy(
            srcs.at[0], src_b.at[pl.ds(s * NSUB, NSUB)], sem_l[s]).wait()
        pltpu.make_async_copy(
            dsts.at[0], dst_b.at[pl.ds(s * ECH, ECH)], sem_l[s]).wait()

    def issue_gathers(s):
        for j in range(NSUB):
            pltpu.async_copy(my_x.at[src_b.at[s * NSUB + j]],
                             vals.at[pl.ds(s * ECH + j * SUB, SUB)], sem_g[s])

    def wait_gathers(s):
        pltpu.make_async_copy(my_x.at[pl.ds(0, ECH)],
                              vals.at[pl.ds(s * ECH, ECH)], sem_g[s]).wait()

    pats = [jnp.reshape(2 * k + half, (16, 1)) for k in range(8)]
    lane16 = lane + 16
    zerov = lane & 0

    def detect2(dA, dB):
        kA = dA & 16383
        kB = dB & 16383
        plsc.store_scatter(scr, [kA], lane)
        plsc.store_scatter(scr, [kB], lane16)
        gA = plsc.load_gather(scr, [kA])
        gB = plsc.load_gather(scr, [kB])
        bad = (gA != lane) | (gB != lane16)
        return jnp.max(jnp.where(bad, jnp.int32(1), jnp.int32(0)))

    def compute(s):
        dbase = s * ECH
        dA0 = dst_b[pl.ds(dbase, 16)]
        dB0 = dst_b[pl.ds(dbase + 16, 16)]
        dup0 = detect2(dA0, dB0)

        def block(g, carry):
            dup, dA, dB = carry
            e0 = dbase + 32 * g
            avA = dA * CPW
            avB = dB * CPW
            flat0 = e0 * CPW + lane
            addrs = []
            vvals = []
            for k in range(16):
                av = avA if k < 8 else avB
                a_k = lax.gather(av, pats[k % 8], _GATHER_DNUMS, (1,),
                                 mode=lax.GatherScatterMode.PROMISE_IN_BOUNDS)
                a_k = a_k | col8
                v_k = plsc.load_gather(vals, [zerov, flat0 + 16 * k])
                addrs.append(a_k)
                vvals.append(v_k)
            dA_n = dst_b[pl.ds(e0 + 32, 16)]
            dB_n = dst_b[pl.ds(e0 + 48, 16)]
            dup_n = detect2(dA_n, dB_n)

            def fast():
                for h in (0, 8):
                    accs = [plsc.load_gather(acc, [addrs[h + k]])
                            for k in range(8)]
                    for k in range(8):
                        plsc.store_scatter(acc, [addrs[h + k]],
                                           jnp.maximum(accs[k], vvals[h + k]))

            def slow():
                for k in range(16):
                    vs = _swap_halves(vvals[k], perm)
                    asw = _swap_halves(addrs[k], perm)
                    merged = jnp.where(addrs[k] == asw,
                                       jnp.maximum(vvals[k], vs), vvals[k])
                    a = plsc.load_gather(acc, [addrs[k]])
                    plsc.store_scatter(acc, [addrs[k]], jnp.maximum(a, merged))

            lax.cond(dup > 0, slow, fast)
            return (dup_n, dA_n, dB_n)

        lax.fori_loop(0, ECH // 32, block, (dup0, dA0, dB0))

    issue_linear(0, 0)
    issue_linear(1, 1)
    wait_linear(0)
    issue_gathers(0)

    def two_chunks(t, carry):
        for s in (0, 1):
            c = 2 * t + s
            wait_gathers(s)
            wait_linear(1 - s)
            issue_gathers(1 - s)
            compute(s)
            issue_linear(jnp.minimum(c + 2, NCH - 1), s)
        return carry
    lax.fori_loop(0, NCH // 2, two_chunks, 0)

    wait_gathers(0)
    wait_linear(1)

    def fix(i, carry):
        v = acc[pl.ds(i * 16, 16)]
        acc[pl.ds(i * 16, 16)] = jnp.where(v == NEG_INF, jnp.float32(0.0), v)
        return carry
    lax.fori_loop(0, ACC // 16, fix, 0, unroll=4)
    pltpu.sync_copy(acc, out.at[wid])


def _sc_agg(x, src, dst):
    xt = x.reshape(N_NODES, NW, CPW).transpose(1, 0, 2)
    srcs = src.reshape(NCH, NSUB, SUB)
    dsts = dst.reshape(NCH, ECH)
    mesh = plsc.VectorSubcoreMesh(core_axis_name="c", subcore_axis_name="s")
    f = pl.kernel(
        _body,
        out_type=jax.ShapeDtypeStruct((NW, ACC), jnp.float32),
        mesh=mesh,
        scratch_types=[
            pltpu.VMEM((2 * NSUB, SUB), jnp.int32),
            pltpu.VMEM((2 * ECH + 48,), jnp.int32),
            pltpu.VMEM((2 * ECH, CPW), jnp.float32),
            pltpu.VMEM((ACC,), jnp.float32),
            pltpu.VMEM((16384,), jnp.int32),
            pltpu.SemaphoreType.DMA,
            pltpu.SemaphoreType.DMA,
            pltpu.SemaphoreType.DMA,
            pltpu.SemaphoreType.DMA,
        ],
        compiler_params=pltpu.CompilerParams(
            needs_layout_passes=False, use_tc_tiling_on_sc=False),
    )
    agg32 = f(xt, srcs, dsts)
    return agg32.reshape(NW, N_NODES, CPW).transpose(1, 0, 2).reshape(
        N_NODES, D_FEAT)


def kernel(x, edge_index):
    src = edge_index[0].astype(jnp.int32)
    dst = edge_index[1].astype(jnp.int32)
    agg = _sc_agg(x, src, dst)
    return jnp.concatenate([x, agg], axis=-1)


# speedup vs baseline: 1.0001x; 1.0001x over previous
import jax
import jax.numpy as jnp
from jax import lax
from jax.experimental import pallas as pl
from jax.experimental.pallas import tpu as pltpu
from jax.experimental.pallas import tpu_sc as plsc

N_NODES = 10000
D_FEAT = 256
N_EDGES = 160000
NC, NS = 2, 16
NW = NC * NS
CPW = D_FEAT // NW
ECH = 640
NCH = N_EDGES // ECH
SUB = 128
NSUB = ECH // SUB
ACC = N_NODES * CPW

NEG_INF = float("-inf")

_GATHER_DNUMS = lax.GatherDimensionNumbers(
    offset_dims=(), collapsed_slice_dims=(0,), start_index_map=(0,))


def _swap_halves(v, perm):
    return lax.gather(v, perm, _GATHER_DNUMS, (1,),
                      mode=lax.GatherScatterMode.PROMISE_IN_BOUNDS)


def _body(xt, srcs, dsts, out, src_b, dst_b, vals, acc, scr,
          sl0, sl1, sg0, sg1):
    wid = lax.axis_index("c") * NS + lax.axis_index("s")
    lane = lax.iota(jnp.int32, 16)
    col8 = lane & 7
    half = lane >> 3
    perm = jnp.reshape(lane ^ 8, (16, 1))

    def init(i, carry):
        acc[pl.ds(i * 16, 16)] = jnp.full((16,), NEG_INF, jnp.float32)
        return carry
    lax.fori_loop(0, ACC // 16, init, 0, unroll=4)

    my_x = xt.at[wid]
    sem_l = (sl0, sl1)
    sem_g = (sg0, sg1)

    def issue_linear(c, s):
        pltpu.async_copy(srcs.at[c], src_b.at[pl.ds(s * NSUB, NSUB)], sem_l[s])
        pltpu.async_copy(dsts.at[c], dst_b.at[pl.ds(s * ECH, ECH)], sem_l[s])

    def wait_linear(s):
        pltpu.make_async_copy(
            srcs.at[0], src_b.at[pl.ds(s * NSUB, NSUB)], sem_l[s]).wait()
        pltpu.make_async_copy(
            dsts.at[0], dst_b.at[pl.ds(s * ECH, ECH)], sem_l[s]).wait()

    def issue_gathers(s):
        for j in range(NSUB):
            pltpu.async_copy(my_x.at[src_b.at[s * NSUB + j]],
                             vals.at[pl.ds(s * ECH + j * SUB, SUB)], sem_g[s])

    def wait_gathers(s):
        pltpu.make_async_copy(my_x.at[pl.ds(0, ECH)],
                              vals.at[pl.ds(s * ECH, ECH)], sem_g[s]).wait()

    pats = [jnp.reshape(2 * k + half, (16, 1)) for k in range(8)]
    lane16 = lane + 16
    zerov = jnp.zeros((16,), jnp.int32)

    def detect2(dA, dB):
        kA = dA & 16383
        kB = dB & 16383
        plsc.store_scatter(scr, [kA], lane)
        plsc.store_scatter(scr, [kB], lane16)
        gA = plsc.load_gather(scr, [kA])
        gB = plsc.load_gather(scr, [kB])
        bad = (gA != lane) | (gB != lane16)
        return jnp.max(jnp.where(bad, jnp.int32(1), jnp.int32(0)))

    def compute(s):
        dbase = s * ECH
        dA0 = dst_b[pl.ds(dbase, 16)]
        dB0 = dst_b[pl.ds(dbase + 16, 16)]
        dup0 = detect2(dA0, dB0)

        def block(g, carry):
            dup, dA, dB = carry
            e0 = dbase + 32 * g
            avA = dA * CPW
            avB = dB * CPW
            flat0 = e0 * CPW + lane
            addrs = []
            vvals = []
            for k in range(16):
                av = avA if k < 8 else avB
                a_k = lax.gather(av, pats[k % 8], _GATHER_DNUMS, (1,),
                                 mode=lax.GatherScatterMode.PROMISE_IN_BOUNDS)
                a_k = a_k | col8
                v_k = plsc.load_gather(vals, [zerov, flat0 + 16 * k])
                addrs.append(a_k)
                vvals.append(v_k)
            dA_n = dst_b[pl.ds(e0 + 32, 16)]
            dB_n = dst_b[pl.ds(e0 + 48, 16)]
            dup_n = detect2(dA_n, dB_n)

            def fast():
                for h in (0, 8):
                    accs = [plsc.load_gather(acc, [addrs[h + k]])
                            for k in range(8)]
                    for k in range(8):
                        plsc.store_scatter(acc, [addrs[h + k]],
                                           jnp.maximum(accs[k], vvals[h + k]))

            def slow():
                for k in range(16):
                    vs = _swap_halves(vvals[k], perm)
                    asw = _swap_halves(addrs[k], perm)
                    merged = jnp.where(addrs[k] == asw,
                                       jnp.maximum(vvals[k], vs), vvals[k])
                    a = plsc.load_gather(acc, [addrs[k]])
                    plsc.store_scatter(acc, [addrs[k]], jnp.maximum(a, merged))

            lax.cond(dup > 0, slow, fast)
            return (dup_n, dA_n, dB_n)

        lax.fori_loop(0, ECH // 32, block, (dup0, dA0, dB0))

    issue_linear(0, 0)
    issue_linear(1, 1)
    wait_linear(0)
    issue_gathers(0)

    def two_chunks(t, carry):
        for s in (0, 1):
            c = 2 * t + s
            wait_gathers(s)
            wait_linear(1 - s)
            issue_gathers(1 - s)
            compute(s)
            issue_linear(jnp.minimum(c + 2, NCH - 1), s)
        return carry
    lax.fori_loop(0, NCH // 2, two_chunks, 0)

    wait_gathers(0)
    wait_linear(1)

    def fix(i, carry):
        v = acc[pl.ds(i * 16, 16)]
        acc[pl.ds(i * 16, 16)] = jnp.where(v == NEG_INF, jnp.float32(0.0), v)
        return carry
    lax.fori_loop(0, ACC // 16, fix, 0, unroll=4)
    pltpu.sync_copy(acc, out.at[wid])


def _sc_agg(x, src, dst):
    xt = x.reshape(N_NODES, NW, CPW).transpose(1, 0, 2)
    srcs = src.reshape(NCH, NSUB, SUB)
    dsts = dst.reshape(NCH, ECH)
    mesh = plsc.VectorSubcoreMesh(core_axis_name="c", subcore_axis_name="s")
    f = pl.kernel(
        _body,
        out_type=jax.ShapeDtypeStruct((NW, ACC), jnp.float32),
        mesh=mesh,
        scratch_types=[
            pltpu.VMEM((2 * NSUB, SUB), jnp.int32),
            pltpu.VMEM((2 * ECH + 48,), jnp.int32),
            pltpu.VMEM((2 * ECH, CPW), jnp.float32),
            pltpu.VMEM((ACC,), jnp.float32),
            pltpu.VMEM((16384,), jnp.int32),
            pltpu.SemaphoreType.DMA,
            pltpu.SemaphoreType.DMA,
            pltpu.SemaphoreType.DMA,
            pltpu.SemaphoreType.DMA,
        ],
        compiler_params=pltpu.CompilerParams(
            needs_layout_passes=False, use_tc_tiling_on_sc=False),
    )
    agg32 = f(xt, srcs, dsts)
    return agg32.reshape(NW, N_NODES, CPW).transpose(1, 0, 2).reshape(
        N_NODES, D_FEAT)


def kernel(x, edge_index):
    src = edge_index[0].astype(jnp.int32)
    dst = edge_index[1].astype(jnp.int32)
    agg = _sc_agg(x, src, dst)
    return jnp.concatenate([x, agg], axis=-1)


# speedup vs baseline: 1.0266x; 1.0265x over previous
import jax
import jax.numpy as jnp
from jax import lax
from jax.experimental import pallas as pl
from jax.experimental.pallas import tpu as pltpu
from jax.experimental.pallas import tpu_sc as plsc

N_NODES = 10000
D_FEAT = 256
N_EDGES = 160000
NC, NS = 2, 16
NW = NC * NS
CPW = D_FEAT // NW
ECH = 640
NCH = N_EDGES // ECH
SUB = 128
NSUB = ECH // SUB
ACC = N_NODES * CPW

NEG_INF = float("-inf")

_GATHER_DNUMS = lax.GatherDimensionNumbers(
    offset_dims=(), collapsed_slice_dims=(0,), start_index_map=(0,))


def _swap_halves(v, perm):
    return lax.gather(v, perm, _GATHER_DNUMS, (1,),
                      mode=lax.GatherScatterMode.PROMISE_IN_BOUNDS)


def _body(xt, srcs, dsts, out, src_b, dst_b, vals, acc, scr,
          sl0, sl1, sg0, sg1):
    wid = lax.axis_index("c") * NS + lax.axis_index("s")
    lane = lax.iota(jnp.int32, 16)
    col8 = lane & 7
    half = lane >> 3
    perm = jnp.reshape(lane ^ 8, (16, 1))

    def init(i, carry):
        acc[pl.ds(i * 16, 16)] = jnp.full((16,), NEG_INF, jnp.float32)
        return carry
    lax.fori_loop(0, ACC // 16, init, 0, unroll=4)

    my_x = xt.at[wid]
    sem_l = (sl0, sl1)
    sem_g = (sg0, sg1)

    def issue_linear(c, s):
        pltpu.async_copy(srcs.at[c], src_b.at[pl.ds(s * NSUB, NSUB)], sem_l[s])
        pltpu.async_copy(dsts.at[c], dst_b.at[pl.ds(s * ECH, ECH)], sem_l[s])

    def wait_linear(s):
        pltpu.make_async_copy(
            srcs.at[0], src_b.at[pl.ds(s * NSUB, NSUB)], sem_l[s]).wait()
        pltpu.make_async_copy(
            dsts.at[0], dst_b.at[pl.ds(s * ECH, ECH)], sem_l[s]).wait()

    def issue_gathers(s):
        for j in range(NSUB):
            pltpu.async_copy(my_x.at[src_b.at[s * NSUB + j]],
                             vals.at[pl.ds(s * ECH + j * SUB, SUB)], sem_g[s])

    def wait_gathers(s):
        pltpu.make_async_copy(my_x.at[pl.ds(0, ECH)],
                              vals.at[pl.ds(s * ECH, ECH)], sem_g[s]).wait()

    pats = [jnp.reshape(2 * k + half, (16, 1)) for k in range(8)]
    lane16 = lane + 16

    def detect2(dA, dB):
        kA = dA & 16383
        kB = dB & 16383
        plsc.store_scatter(scr, [kA], lane)
        plsc.store_scatter(scr, [kB], lane16)
        gA = plsc.load_gather(scr, [kA])
        gB = plsc.load_gather(scr, [kB])
        bad = (gA != lane) | (gB != lane16)
        return jnp.max(jnp.where(bad, jnp.int32(1), jnp.int32(0)))

    def compute(s):
        dbase = s * ECH
        dA0 = dst_b[pl.ds(dbase, 16)]
        dB0 = dst_b[pl.ds(dbase + 16, 16)]
        dup0 = detect2(dA0, dB0)

        def block(g, carry):
            dup, dA, dB = carry
            e0 = dbase + 32 * g
            avA = dA * CPW
            avB = dB * CPW
            addrs = []
            vvals = []
            for k in range(16):
                av = avA if k < 8 else avB
                a_k = lax.gather(av, pats[k % 8], _GATHER_DNUMS, (1,),
                                 mode=lax.GatherScatterMode.PROMISE_IN_BOUNDS)
                a_k = a_k | col8
                rows = (e0 + 2 * k) + half
                v_k = plsc.load_gather(vals, [rows, col8])
                addrs.append(a_k)
                vvals.append(v_k)
            dA_n = dst_b[pl.ds(e0 + 32, 16)]
            dB_n = dst_b[pl.ds(e0 + 48, 16)]
            dup_n = detect2(dA_n, dB_n)

            def fast():
                for h in (0, 8):
                    accs = [plsc.load_gather(acc, [addrs[h + k]])
                            for k in range(8)]
                    for k in range(8):
                        plsc.store_scatter(acc, [addrs[h + k]],
                                           jnp.maximum(accs[k], vvals[h + k]))

            def slow():
                for k in range(16):
                    vs = _swap_halves(vvals[k], perm)
                    asw = _swap_halves(addrs[k], perm)
                    merged = jnp.where(addrs[k] == asw,
                                       jnp.maximum(vvals[k], vs), vvals[k])
                    a = plsc.load_gather(acc, [addrs[k]])
                    plsc.store_scatter(acc, [addrs[k]], jnp.maximum(a, merged))

            lax.cond(dup > 0, slow, fast)
            return (dup_n, dA_n, dB_n)

        lax.fori_loop(0, ECH // 32, block, (dup0, dA0, dB0))

    issue_linear(0, 0)
    issue_linear(1, 1)
    wait_linear(0)
    issue_gathers(0)

    def two_chunks(t, carry):
        for s in (0, 1):
            c = 2 * t + s
            wait_gathers(s)
            wait_linear(1 - s)
            issue_gathers(1 - s)
            compute(s)
            issue_linear(jnp.minimum(c + 2, NCH - 1), s)
        return carry
    lax.fori_loop(0, NCH // 2, two_chunks, 0)

    wait_gathers(0)
    wait_linear(1)

    def fix(i, carry):
        v = acc[pl.ds(i * 16, 16)]
        acc[pl.ds(i * 16, 16)] = jnp.where(v == NEG_INF, jnp.float32(0.0), v)
        return carry
    lax.fori_loop(0, ACC // 16, fix, 0, unroll=4)
    pltpu.sync_copy(acc, out.at[wid])


def _sc_agg(x, src, dst):
    xt = x.reshape(N_NODES, NW, CPW).transpose(1, 0, 2)
    srcs = src.reshape(NCH, NSUB, SUB)
    dsts = dst.reshape(NCH, ECH)
    mesh = plsc.VectorSubcoreMesh(core_axis_name="c", subcore_axis_name="s")
    f = pl.kernel(
        _body,
        out_type=jax.ShapeDtypeStruct((NW, ACC), jnp.float32),
        mesh=mesh,
        scratch_types=[
            pltpu.VMEM((2 * NSUB, SUB), jnp.int32),
            pltpu.VMEM((2 * ECH + 48,), jnp.int32),
            pltpu.VMEM((2 * ECH, CPW), jnp.float32),
            pltpu.VMEM((ACC,), jnp.float32),
            pltpu.VMEM((16384,), jnp.int32),
            pltpu.SemaphoreType.DMA,
            pltpu.SemaphoreType.DMA,
            pltpu.SemaphoreType.DMA,
            pltpu.SemaphoreType.DMA,
        ],
        compiler_params=pltpu.CompilerParams(
            needs_layout_passes=False, use_tc_tiling_on_sc=False),
    )
    agg32 = f(xt, srcs, dsts)
    return agg32.reshape(NW, N_NODES, CPW).transpose(1, 0, 2).reshape(
        N_NODES, D_FEAT)


def kernel(x, edge_index):
    src = edge_index[0].astype(jnp.int32)
    dst = edge_index[1].astype(jnp.int32)
    agg = _sc_agg(x, src, dst)
    return jnp.concatenate([x, agg], axis=-1)


# speedup vs baseline: 1.2281x; 1.1962x over previous
import jax
import jax.numpy as jnp
from jax import lax
from jax.experimental import pallas as pl
from jax.experimental.pallas import tpu as pltpu
from jax.experimental.pallas import tpu_sc as plsc

N_NODES = 10000
D_FEAT = 256
N_EDGES = 160000
NC, NS = 2, 16
NW = NC * NS
CPW = D_FEAT // NW
ECH = 640
NCH = N_EDGES // ECH
SUB = 128
NSUB = ECH // SUB
ACC = N_NODES * CPW

NEG_INF = float("-inf")

_GATHER_DNUMS = lax.GatherDimensionNumbers(
    offset_dims=(), collapsed_slice_dims=(0,), start_index_map=(0,))


def _swap_halves(v, perm):
    return lax.gather(v, perm, _GATHER_DNUMS, (1,),
                      mode=lax.GatherScatterMode.PROMISE_IN_BOUNDS)


def _body(x2, srcs, dsts, out, src_b, gsrc_b, dst_b, vals, acc, scr,
          sl0, sl1, sg0, sg1):
    wid = lax.axis_index("c") * NS + lax.axis_index("s")
    lane = lax.iota(jnp.int32, 16)
    col8 = lane & 7
    half = lane >> 3
    perm = jnp.reshape(lane ^ 8, (16, 1))

    def init(i, carry):
        acc[pl.ds(i * 16, 16)] = jnp.full((16,), NEG_INF, jnp.float32)
        return carry
    lax.fori_loop(0, ACC // 16, init, 0, unroll=4)

    sem_l = (sl0, sl1)
    sem_g = (sg0, sg1)

    def issue_linear(c, s):
        pltpu.async_copy(srcs.at[c], src_b.at[pl.ds(s * ECH, ECH)], sem_l[s])
        pltpu.async_copy(dsts.at[c], dst_b.at[pl.ds(s * ECH, ECH)], sem_l[s])

    def wait_linear(s):
        pltpu.make_async_copy(
            srcs.at[0], src_b.at[pl.ds(s * ECH, ECH)], sem_l[s]).wait()
        pltpu.make_async_copy(
            dsts.at[0], dst_b.at[pl.ds(s * ECH, ECH)], sem_l[s]).wait()

    def scale_srcs(s):
        def scale(i, carry):
            off = s * ECH + i * 16
            v = src_b[pl.ds(off, 16)]
            gsrc_b[pl.ds(off, 16)] = (v << 5) + wid
            return carry
        lax.fori_loop(0, ECH // 16, scale, 0, unroll=4)

    def issue_gathers(s):
        for j in range(NSUB):
            pltpu.async_copy(
                x2.at[gsrc_b.at[pl.ds(s * ECH + j * SUB, SUB)]],
                vals.at[pl.ds(s * ECH + j * SUB, SUB)], sem_g[s])

    def wait_gathers(s):
        pltpu.make_async_copy(x2.at[pl.ds(0, ECH)],
                              vals.at[pl.ds(s * ECH, ECH)], sem_g[s]).wait()

    pats = [jnp.reshape(2 * k + half, (16, 1)) for k in range(8)]
    lane16 = lane + 16

    def detect2(dA, dB):
        kA = dA & 16383
        kB = dB & 16383
        plsc.store_scatter(scr, [kA], lane)
        plsc.store_scatter(scr, [kB], lane16)
        gA = plsc.load_gather(scr, [kA])
        gB = plsc.load_gather(scr, [kB])
        bad = (gA != lane) | (gB != lane16)
        return jnp.max(jnp.where(bad, jnp.int32(1), jnp.int32(0)))

    def compute(s):
        dbase = s * ECH
        dA0 = dst_b[pl.ds(dbase, 16)]
        dB0 = dst_b[pl.ds(dbase + 16, 16)]
        dup0 = detect2(dA0, dB0)

        def block(g, carry):
            dup, dA, dB = carry
            e0 = dbase + 32 * g
            avA = dA * CPW
            avB = dB * CPW
            addrs = []
            vvals = []
            for k in range(16):
                av = avA if k < 8 else avB
                a_k = lax.gather(av, pats[k % 8], _GATHER_DNUMS, (1,),
                                 mode=lax.GatherScatterMode.PROMISE_IN_BOUNDS)
                a_k = a_k | col8
                rows = (e0 + 2 * k) + half
                v_k = plsc.load_gather(vals, [rows, col8])
                addrs.append(a_k)
                vvals.append(v_k)
            dA_n = dst_b[pl.ds(e0 + 32, 16)]
            dB_n = dst_b[pl.ds(e0 + 48, 16)]
            dup_n = detect2(dA_n, dB_n)

            def fast():
                for h in (0, 8):
                    accs = [plsc.load_gather(acc, [addrs[h + k]])
                            for k in range(8)]
                    for k in range(8):
                        plsc.store_scatter(acc, [addrs[h + k]],
                                           jnp.maximum(accs[k], vvals[h + k]))

            def slow():
                for k in range(16):
                    vs = _swap_halves(vvals[k], perm)
                    asw = _swap_halves(addrs[k], perm)
                    merged = jnp.where(addrs[k] == asw,
                                       jnp.maximum(vvals[k], vs), vvals[k])
                    a = plsc.load_gather(acc, [addrs[k]])
                    plsc.store_scatter(acc, [addrs[k]], jnp.maximum(a, merged))

            lax.cond(dup > 0, slow, fast)
            return (dup_n, dA_n, dB_n)

        lax.fori_loop(0, ECH // 32, block, (dup0, dA0, dB0))

    issue_linear(0, 0)
    issue_linear(1, 1)
    wait_linear(0)
    scale_srcs(0)
    issue_gathers(0)

    def two_chunks(t, carry):
        for s in (0, 1):
            c = 2 * t + s
            wait_gathers(s)
            wait_linear(1 - s)
            scale_srcs(1 - s)
            issue_gathers(1 - s)
            compute(s)
            issue_linear(jnp.minimum(c + 2, NCH - 1), s)
        return carry
    lax.fori_loop(0, NCH // 2, two_chunks, 0)

    wait_gathers(0)
    wait_linear(1)

    def fix(i, carry):
        v = acc[pl.ds(i * 16, 16)]
        acc[pl.ds(i * 16, 16)] = jnp.where(v == NEG_INF, jnp.float32(0.0), v)
        return carry
    lax.fori_loop(0, ACC // 16, fix, 0, unroll=4)
    pltpu.sync_copy(acc, out.at[wid])


def _sc_agg(x, src, dst):
    x2 = x.reshape(N_NODES * NW, CPW)
    srcs = src.reshape(NCH, ECH)
    dsts = dst.reshape(NCH, ECH)
    mesh = plsc.VectorSubcoreMesh(core_axis_name="c", subcore_axis_name="s")
    f = pl.kernel(
        _body,
        out_type=jax.ShapeDtypeStruct((NW, ACC), jnp.float32),
        mesh=mesh,
        scratch_types=[
            pltpu.VMEM((2 * ECH,), jnp.int32),
            pltpu.VMEM((2 * ECH,), jnp.int32),
            pltpu.VMEM((2 * ECH + 48,), jnp.int32),
            pltpu.VMEM((2 * ECH, CPW), jnp.float32),
            pltpu.VMEM((ACC,), jnp.float32),
            pltpu.VMEM((16384,), jnp.int32),
            pltpu.SemaphoreType.DMA,
            pltpu.SemaphoreType.DMA,
            pltpu.SemaphoreType.DMA,
            pltpu.SemaphoreType.DMA,
        ],
        compiler_params=pltpu.CompilerParams(
            needs_layout_passes=False, use_tc_tiling_on_sc=False),
    )
    agg32 = f(x2, srcs, dsts)
    return agg32.reshape(NW, N_NODES, CPW).transpose(1, 0, 2).reshape(
        N_NODES, D_FEAT)


def kernel(x, edge_index):
    src = edge_index[0].astype(jnp.int32)
    dst = edge_index[1].astype(jnp.int32)
    agg = _sc_agg(x, src, dst)
    return jnp.concatenate([x, agg], axis=-1)
